# Initial kernel scaffold; baseline (speedup 1.0000x reference)
import jax
import jax.numpy as jnp
from jax.experimental import pallas as pl


def kernel(voxel_coords, sdf, cube_idx, resolution, deform, beta, alpha, gamma):
    raise NotImplementedError("write your pallas kernel here")


# speedup vs baseline: 8.9532x; 8.9532x over previous
import functools

import numpy as np
import jax
import jax.numpy as jnp
from jax import lax
from jax.experimental import pallas as pl
from jax.experimental.pallas import tpu as pltpu
from jax.experimental.pallas import tpu_sc as plsc

_EA = np.array([0, 1, 4, 0, 2, 3, 6, 2, 2, 3, 7, 6], dtype=np.int32)
_EB = np.array([1, 5, 5, 4, 3, 7, 7, 6, 0, 1, 5, 4], dtype=np.int32)


def _sel_matrix():
    S = np.zeros((32, 128), dtype=np.float32)
    for e in range(12):
        S[4 * _EA[e] + 3, e] = 1.0
        S[4 * _EB[e] + 3, 12 + e] = 1.0
        S[4 * _EA[e] + 0, 24 + e] = 1.0
        S[4 * _EB[e] + 0, 36 + e] = 1.0
        S[4 * _EA[e] + 1, 48 + e] = 1.0
        S[4 * _EB[e] + 1, 60 + e] = 1.0
        S[4 * _EA[e] + 2, 72 + e] = 1.0
        S[4 * _EB[e] + 2, 84 + e] = 1.0
    for c in range(8):
        S[4 * c + 3, 96 + c] = 1.0
        S[4 * c + 0, 104 + c] = 1.0
        S[4 * c + 1, 112 + c] = 1.0
        S[4 * c + 2, 120 + c] = 1.0
    return S


_BIGSEL = _sel_matrix()
_ISPOS = np.ones((1, 32), dtype=np.float32)
_ISPOS[0, 3::4] = 0.0


def _scan_kernel(idx_hbm, w_out, w_v, idxwin_v, K, RANGE, WIN):
    h = lax.axis_index("c")
    r = lax.axis_index("s")
    lo = r * RANGE
    KH = K // 2
    nwin = KH // WIN
    nv_init = RANGE // 16

    def init_body(i, _):
        w_v[pl.ds(i * 16, 16)] = jnp.full((16,), -1, jnp.int32)
        return _

    lax.fori_loop(0, nv_init, init_body, 0, unroll=4)

    iota = lax.iota(jnp.int32, 16)

    def win_body(w, _):
        base = h * KH + w * WIN
        pltpu.sync_copy(idx_hbm.at[pl.ds(base, WIN)], idxwin_v)
        kv0 = base + iota

        def v_body(i, _):
            iv = idxwin_v[pl.ds(i * 16, 16)]
            off = iv - lo
            m = (off >= 0) & (off < RANGE)
            plsc.store_scatter(w_v, [off], kv0 + i * 16, mask=m)
            return _

        lax.fori_loop(0, WIN // 16, v_body, 0, unroll=8)
        return _

    lax.fori_loop(0, nwin, win_body, 0)
    pltpu.sync_copy(w_v, w_out.at[pl.ds((h * 16 + r) * RANGE, RANGE)])


def _merge_kernel(w_halves, vox_hbm, sdf_hbm, defs_hbm, table_out,
                  w0_v, w1_v, kbuf_v, iv_v, voxrows_v, sdfwin_v, defwin_v,
                  out_v, sem, RANGE, RB2, WIN2, NV):
    t = lax.axis_index("s") * 2 + lax.axis_index("c")
    r = t // 2
    iota = lax.iota(jnp.int32, 16)
    zeros16 = jnp.zeros((16,), jnp.int32)
    ones16 = jnp.full((16,), 1, jnp.int32)
    twos16 = jnp.full((16,), 2, jnp.int32)
    threes16 = jnp.full((16,), 3, jnp.int32)

    def win_body(w, _):
        off_in_range = (t % 2) * RB2 + w * WIN2
        row_base = r * RANGE + off_in_range
        pltpu.sync_copy(
            w_halves.at[pl.ds(r * (RANGE // 2) * 2 + off_in_range, WIN2)], w0_v)
        pltpu.sync_copy(
            w_halves.at[pl.ds((16 + r) * (RANGE // 2) * 2 + off_in_range, WIN2)],
            w1_v)
        pltpu.sync_copy(sdf_hbm.at[pl.ds(row_base, WIN2)], sdfwin_v)
        pltpu.sync_copy(defs_hbm.at[pl.ds(row_base, WIN2)], defwin_v)

        def merge_body(i, _):
            k = jnp.maximum(w0_v[pl.ds(i * 16, 16)], w1_v[pl.ds(i * 16, 16)])
            kbuf_v[pl.ds(i * 16, 16)] = k
            i_safe = jnp.where(k >= 0, lax.shift_right_logical(k, 3), 0)
            iv_v[pl.ds(i * 16, 16)] = i_safe
            return _

        lax.fori_loop(0, NV, merge_body, 0, unroll=8)
        pltpu.async_copy(vox_hbm.at[iv_v], voxrows_v, sem).wait()

        def build_body(i, _):
            rows = i * 16 + iota
            k = kbuf_v[pl.ds(i * 16, 16)]
            valid = k >= 0
            c = k & 7
            vx = plsc.load_gather(voxrows_v, [rows, zeros16])
            vy = plsc.load_gather(voxrows_v, [rows, ones16])
            vz = plsc.load_gather(voxrows_v, [rows, twos16])
            px = jnp.where(valid, (vx + (c & 1)).astype(jnp.float32), 0.0)
            py = jnp.where(valid,
                           (vy + (lax.shift_right_logical(k, 1) & 1)).astype(jnp.float32), 0.0)
            pz = jnp.where(valid,
                           (vz + (lax.shift_right_logical(k, 2) & 1)).astype(jnp.float32), 0.0)
            dx = plsc.load_gather(defwin_v, [rows, zeros16])
            dy = plsc.load_gather(defwin_v, [rows, ones16])
            dz = plsc.load_gather(defwin_v, [rows, twos16])
            s = sdfwin_v[pl.ds(i * 16, 16)]
            plsc.store_scatter(out_v, [rows, zeros16], px + dx)
            plsc.store_scatter(out_v, [rows, ones16], py + dy)
            plsc.store_scatter(out_v, [rows, twos16], pz + dz)
            plsc.store_scatter(out_v, [rows, threes16], s)
            return _

        lax.fori_loop(0, NV, build_body, 0, unroll=4)
        pltpu.sync_copy(out_v, table_out.at[pl.ds(row_base, WIN2)])
        return _

    lax.fori_loop(0, RB2 // WIN2, win_body, 0)


def _gather_kernel(idx_hbm, table_hbm, g_out, idxwin_v, rows_v, sem, KC, WC):
    t = lax.axis_index("s") * 2 + lax.axis_index("c")

    def win_body(w, _):
        base = t * KC + w * WC
        pltpu.sync_copy(idx_hbm.at[pl.ds(base, WC)], idxwin_v)
        pltpu.async_copy(table_hbm.at[idxwin_v], rows_v, sem).wait()
        pltpu.sync_copy(rows_v, g_out.at[pl.ds(base, WC)])
        return _

    lax.fori_loop(0, KC // WC, win_body, 0)


def _dense_kernel(g_ref, beta_ref, alpha_ref, gamma_ref, sel_ref, a_ref,
                  b_ref, out_ref):
    g = g_ref[...] * a_ref[0:1, :] + b_ref[0:1, :]
    cols = jax.lax.dot(g, sel_ref[...], precision=jax.lax.Precision.HIGHEST,
                       preferred_element_type=jnp.float32)
    sa = cols[:, 0:12]
    sb = cols[:, 12:24]
    pax = cols[:, 24:36]
    pbx = cols[:, 36:48]
    pay = cols[:, 48:60]
    pby = cols[:, 60:72]
    paz = cols[:, 72:84]
    pbz = cols[:, 84:96]
    s8 = cols[:, 96:104]
    px = cols[:, 104:112]
    py = cols[:, 112:120]
    pz = cols[:, 120:128]

    occ = (s8 < 0).astype(jnp.float32)
    occ_sum = jnp.sum(occ, axis=1, keepdims=True)
    surf = ((occ_sum > 0) & (occ_sum < 8)).astype(jnp.float32)

    ws_ = 0.99
    beta_n = jnp.tanh(beta_ref[...]) * ws_ + 1.0
    alpha_n = jnp.tanh(alpha_ref[...]) * ws_ + 1.0
    gm = gamma_ref[...]
    gamma_n = (1.0 / (1.0 + jnp.exp(-gm))) * ws_ + (1.0 - ws_) / 2.0

    denom = sa - sb
    denom = jnp.where(jnp.abs(denom) < 1e-8, 1e-8, denom)
    t = jnp.clip(sa / denom, 0.0, 1.0)
    crossing = ((sa * sb) < 0).astype(jnp.float32)
    bw = beta_n * crossing + 1e-6
    bw_sum = jnp.sum(bw, axis=1, keepdims=True)

    uex = pax + t * (pbx - pax)
    uey = pay + t * (pby - pay)
    uez = paz + t * (pbz - paz)
    vex = jnp.sum(bw * uex, axis=1, keepdims=True) / bw_sum
    vey = jnp.sum(bw * uey, axis=1, keepdims=True) / bw_sum
    vez = jnp.sum(bw * uez, axis=1, keepdims=True) / bw_sum

    a_sum = jnp.sum(alpha_n, axis=1, keepdims=True)
    vcx = jnp.sum(alpha_n * px, axis=1, keepdims=True) / a_sum
    vcy = jnp.sum(alpha_n * py, axis=1, keepdims=True) / a_sum
    vcz = jnp.sum(alpha_n * pz, axis=1, keepdims=True) / a_sum

    vdx = gamma_n * vex + (1.0 - gamma_n) * vcx
    vdy = gamma_n * vey + (1.0 - gamma_n) * vcy
    vdz = gamma_n * vez + (1.0 - gamma_n) * vcz
    out_ref[...] = jnp.concatenate([vdx, vdy, vdz], axis=1) * surf


def kernel(voxel_coords, sdf, cube_idx, resolution, deform, beta, alpha, gamma):
    N_ = cube_idx.shape[0]
    M_ = sdf.shape[0]
    K = N_ * 8

    RB2 = ((M_ + 32 * 16 - 1) // (32 * 16)) * 16
    MP = RB2 * 32
    RANGE = MP // 16
    WIN = 4000
    WIN2 = 2344
    KC = K // 32
    WC = 10000

    ws = 2.0 / jnp.asarray(resolution, jnp.float32)
    idx_flat = cube_idx.reshape(-1)
    sdf_p = jnp.pad(sdf, (0, MP - M_))
    defs = deform * (jnp.asarray(resolution, jnp.float32) * 0.5)
    defs_p = jnp.pad(defs, ((0, MP - M_), (0, 0)))

    mesh = plsc.VectorSubcoreMesh(core_axis_name="c", subcore_axis_name="s")

    scan = functools.partial(
        pl.kernel,
        mesh=mesh,
        compiler_params=pltpu.CompilerParams(needs_layout_passes=False, use_tc_tiling_on_sc=False),
        out_type=jax.ShapeDtypeStruct((2 * 16 * RANGE,), jnp.int32),
        scratch_types=[
            pltpu.VMEM((RANGE,), jnp.int32),
            pltpu.VMEM((WIN,), jnp.int32),
        ],
    )(functools.partial(_scan_kernel, K=K, RANGE=RANGE, WIN=WIN))
    w_halves = scan(idx_flat)

    merge = functools.partial(
        pl.kernel,
        mesh=mesh,
        compiler_params=pltpu.CompilerParams(needs_layout_passes=False, use_tc_tiling_on_sc=False),
        out_type=jax.ShapeDtypeStruct((MP, 4), jnp.float32),
        scratch_types=[
            pltpu.VMEM((WIN2,), jnp.int32),
            pltpu.VMEM((WIN2,), jnp.int32),
            pltpu.VMEM((WIN2,), jnp.int32),
            pltpu.VMEM((WIN2,), jnp.int32),
            pltpu.VMEM((WIN2, 4), jnp.int32),
            pltpu.VMEM((WIN2,), jnp.float32),
            pltpu.VMEM((WIN2, 3), jnp.float32),
            pltpu.VMEM((WIN2, 4), jnp.float32),
            pltpu.SemaphoreType.DMA,
        ],
    )(functools.partial(_merge_kernel, RANGE=RB2 * 2, RB2=RB2, WIN2=WIN2,
                        NV=WIN2 // 16))
    table = merge(w_halves, jnp.pad(voxel_coords, ((0, 0), (0, 1))), sdf_p, defs_p)

    gather = functools.partial(
        pl.kernel,
        mesh=mesh,
        compiler_params=pltpu.CompilerParams(needs_layout_passes=False, use_tc_tiling_on_sc=False),
        out_type=jax.ShapeDtypeStruct((K, 4), jnp.float32),
        scratch_types=[
            pltpu.VMEM((WC,), jnp.int32),
            pltpu.VMEM((WC, 4), jnp.float32),
            pltpu.SemaphoreType.DMA,
        ],
    )(functools.partial(_gather_kernel, KC=KC, WC=WC))
    g = gather(idx_flat, table)

    B = 2000
    out = pl.pallas_call(
        _dense_kernel,
        grid=(N_ // B,),
        in_specs=[
            pl.BlockSpec((B, 32), lambda i: (i, 0)),
            pl.BlockSpec((B, 12), lambda i: (i, 0)),
            pl.BlockSpec((B, 8), lambda i: (i, 0)),
            pl.BlockSpec((B, 1), lambda i: (i, 0)),
            pl.BlockSpec((32, 128), lambda i: (0, 0)),
            pl.BlockSpec((8, 32), lambda i: (0, 0)),
            pl.BlockSpec((8, 32), lambda i: (0, 0)),
        ],
        out_specs=pl.BlockSpec((B, 3), lambda i: (i, 0)),
        out_shape=jax.ShapeDtypeStruct((N_, 3), jnp.float32),
    )(g.reshape(N_, 32), beta, alpha, gamma.reshape(N_, 1),
      jnp.asarray(_BIGSEL),
      jnp.broadcast_to(jnp.asarray(_ISPOS) * ws + (1.0 - jnp.asarray(_ISPOS)),
                       (8, 32)),
      jnp.broadcast_to(jnp.asarray(_ISPOS) * (0.5 * ws - 1.0), (8, 32)))
    return out


# speedup vs baseline: 20.2415x; 2.2608x over previous
import functools

import jax
import jax.numpy as jnp
from jax import lax
from jax.experimental import pallas as pl
from jax.experimental.pallas import tpu as pltpu
from jax.experimental.pallas import tpu_sc as plsc

_EA = (0, 1, 4, 0, 2, 3, 6, 2, 2, 3, 7, 6)
_EB = (1, 5, 5, 4, 3, 7, 7, 6, 0, 1, 5, 4)


def _scan_kernel(idx_hbm, w_out, w_v, idxwin_v, N, RANGE, WIN):
    h = lax.axis_index("c")
    r = lax.axis_index("s")
    lo = r * RANGE
    KH = N * 4
    nwin = KH // WIN

    def init_body(i, _):
        w_v[pl.ds(i * 16, 16)] = jnp.full((16,), -1, jnp.int32)
        return _

    lax.fori_loop(0, RANGE // 16, init_body, 0, unroll=4)

    iota = lax.iota(jnp.int32, 16)

    def win_body(w, _):
        base = h * KH + w * WIN
        pltpu.sync_copy(idx_hbm.at[pl.ds(base, WIN)], idxwin_v)
        c = base // N
        inplane = base - c * N
        kv8 = (inplane + iota) * 8 + c

        def v_body(i, _):
            iv = idxwin_v[pl.ds(i * 16, 16)]
            off = iv - lo
            m = (off >= 0) & (off < RANGE)
            cur = plsc.load_gather(w_v, [off], mask=m)
            kv = kv8 + i * 128
            m2 = m & (kv > cur)
            plsc.store_scatter(w_v, [off], kv, mask=m2)
            return _

        lax.fori_loop(0, WIN // 16, v_body, 0, unroll=8)
        return _

    lax.fori_loop(0, nwin, win_body, 0)
    pltpu.sync_copy(w_v, w_out.at[pl.ds((h * 16 + r) * RANGE, RANGE)])


def _merge_kernel(w_halves, voxf_hbm, sdf_hbm, defst_hbm, table_out,
                  w0_v, w1_v, kbuf_v, iv_v, voxcols_v, sdfwin_v, defwin_v,
                  out_v, sem, N, RANGE, RB2, WIN2, NV):
    t = lax.axis_index("s") * 2 + lax.axis_index("c")
    r = t // 2
    iota = lax.iota(jnp.int32, 16)
    zeros16 = jnp.zeros((16,), jnp.int32)
    ones16 = jnp.full((16,), 1, jnp.int32)
    twos16 = jnp.full((16,), 2, jnp.int32)
    threes16 = jnp.full((16,), 3, jnp.int32)

    def win_body(w, _):
        off_in_range = (t % 2) * RB2 + w * WIN2
        row_base = r * RANGE + off_in_range
        pltpu.sync_copy(w_halves.at[pl.ds(r * RANGE + off_in_range, WIN2)],
                        w0_v)
        pltpu.sync_copy(
            w_halves.at[pl.ds((16 + r) * RANGE + off_in_range, WIN2)], w1_v)
        pltpu.sync_copy(sdf_hbm.at[pl.ds(row_base, WIN2)], sdfwin_v)
        for comp in range(3):
            pltpu.sync_copy(defst_hbm.at[comp, pl.ds(row_base, WIN2)],
                            defwin_v.at[pl.ds(comp * WIN2, WIN2)])

        def merge_body(i, _):
            k = jnp.maximum(w0_v[pl.ds(i * 16, 16)], w1_v[pl.ds(i * 16, 16)])
            kbuf_v[pl.ds(i * 16, 16)] = k
            i_safe = jnp.where(k >= 0, lax.shift_right_logical(k, 3), 0)
            iv_v[pl.ds(i * 16, 16)] = i_safe
            iv_v[pl.ds(WIN2 + i * 16, 16)] = i_safe + N
            iv_v[pl.ds(2 * WIN2 + i * 16, 16)] = i_safe + 2 * N
            return _

        lax.fori_loop(0, NV, merge_body, 0, unroll=8)
        pltpu.async_copy(voxf_hbm.at[iv_v], voxcols_v, sem).wait()

        def build_body(i, _):
            rows = i * 16 + iota
            k = kbuf_v[pl.ds(i * 16, 16)]
            valid = k >= 0
            vx = voxcols_v[pl.ds(i * 16, 16)]
            vy = voxcols_v[pl.ds(WIN2 + i * 16, 16)]
            vz = voxcols_v[pl.ds(2 * WIN2 + i * 16, 16)]
            px = jnp.where(valid, (vx + (k & 1)).astype(jnp.float32), 0.0)
            py = jnp.where(
                valid,
                (vy + (lax.shift_right_logical(k, 1) & 1)).astype(jnp.float32),
                0.0)
            pz = jnp.where(
                valid,
                (vz + (lax.shift_right_logical(k, 2) & 1)).astype(jnp.float32),
                0.0)
            dx = defwin_v[pl.ds(i * 16, 16)]
            dy = defwin_v[pl.ds(WIN2 + i * 16, 16)]
            dz = defwin_v[pl.ds(2 * WIN2 + i * 16, 16)]
            s = sdfwin_v[pl.ds(i * 16, 16)]
            plsc.store_scatter(out_v, [rows, zeros16], px + dx)
            plsc.store_scatter(out_v, [rows, ones16], py + dy)
            plsc.store_scatter(out_v, [rows, twos16], pz + dz)
            plsc.store_scatter(out_v, [rows, threes16], s)
            return _

        lax.fori_loop(0, NV, build_body, 0, unroll=4)
        pltpu.sync_copy(out_v, table_out.at[pl.ds(row_base, WIN2)])
        return _

    lax.fori_loop(0, RB2 // WIN2, win_body, 0)


def _gather_kernel(idx_hbm, table_hbm, g_out, idxwin_v, rows_v,
                   t0_v, t1_v, t2_v, t3_v, sem, KC, WC):
    t = lax.axis_index("s") * 2 + lax.axis_index("c")
    iota = lax.iota(jnp.int32, 16)
    comps = (jnp.zeros((16,), jnp.int32), jnp.full((16,), 1, jnp.int32),
             jnp.full((16,), 2, jnp.int32), jnp.full((16,), 3, jnp.int32))
    tbufs = (t0_v, t1_v, t2_v, t3_v)

    def win_body(w, _):
        base = t * KC + w * WC
        pltpu.sync_copy(idx_hbm.at[pl.ds(base, WC)], idxwin_v)
        pltpu.async_copy(table_hbm.at[idxwin_v], rows_v, sem).wait()

        def tr_body(i, _):
            rows = i * 16 + iota
            for comp in range(4):
                tbufs[comp][pl.ds(i * 16, 16)] = plsc.load_gather(
                    rows_v, [rows, comps[comp]])
            return _

        lax.fori_loop(0, WC // 16, tr_body, 0, unroll=4)
        for comp in range(4):
            pltpu.sync_copy(tbufs[comp], g_out.at[comp, pl.ds(base, WC)])
        return _

    lax.fori_loop(0, KC // WC, win_body, 0)


def _dense_kernel(g4_ref, bt_ref, at_ref, gm_ref, ws_ref, out_ref):
    ws = ws_ref[0, 0]
    s = [g4_ref[24 + c, :] for c in range(8)]
    px = [(g4_ref[0 + c, :] + 0.5) * ws - 1.0 for c in range(8)]
    py = [(g4_ref[8 + c, :] + 0.5) * ws - 1.0 for c in range(8)]
    pz = [(g4_ref[16 + c, :] + 0.5) * ws - 1.0 for c in range(8)]

    occ_sum = sum((sc < 0).astype(jnp.float32) for sc in s)
    surf = ((occ_sum > 0) & (occ_sum < 8)).astype(jnp.float32)

    wsc = 0.99
    alpha_n = [jnp.tanh(at_ref[c, :]) * wsc + 1.0 for c in range(8)]
    gm = gm_ref[0, :]
    gamma_n = (1.0 / (1.0 + jnp.exp(-gm))) * wsc + (1.0 - wsc) / 2.0

    nx = jnp.zeros_like(gm)
    ny = jnp.zeros_like(gm)
    nz = jnp.zeros_like(gm)
    bw_sum = jnp.zeros_like(gm)
    for e in range(12):
        beta_n = jnp.tanh(bt_ref[e, :]) * wsc + 1.0
        sa = s[_EA[e]]
        sb = s[_EB[e]]
        denom = sa - sb
        denom = jnp.where(jnp.abs(denom) < 1e-8, 1e-8, denom)
        te = jnp.clip(sa / denom, 0.0, 1.0)
        crossing = ((sa * sb) < 0).astype(jnp.float32)
        bw = beta_n * crossing + 1e-6
        bw_sum = bw_sum + bw
        pax, pbx = px[_EA[e]], px[_EB[e]]
        pay, pby = py[_EA[e]], py[_EB[e]]
        paz, pbz = pz[_EA[e]], pz[_EB[e]]
        nx = nx + bw * (pax + te * (pbx - pax))
        ny = ny + bw * (pay + te * (pby - pay))
        nz = nz + bw * (paz + te * (pbz - paz))
    vex = nx / bw_sum
    vey = ny / bw_sum
    vez = nz / bw_sum

    a_sum = sum(alpha_n)
    vcx = sum(alpha_n[c] * px[c] for c in range(8)) / a_sum
    vcy = sum(alpha_n[c] * py[c] for c in range(8)) / a_sum
    vcz = sum(alpha_n[c] * pz[c] for c in range(8)) / a_sum

    out_ref[0, :] = (gamma_n * vex + (1.0 - gamma_n) * vcx) * surf
    out_ref[1, :] = (gamma_n * vey + (1.0 - gamma_n) * vcy) * surf
    out_ref[2, :] = (gamma_n * vez + (1.0 - gamma_n) * vcz) * surf


def kernel(voxel_coords, sdf, cube_idx, resolution, deform, beta, alpha, gamma):
    N_ = cube_idx.shape[0]
    M_ = sdf.shape[0]
    K = N_ * 8

    RB2 = ((M_ + 32 * 16 - 1) // (32 * 16)) * 16
    MP = RB2 * 32
    RANGE = MP // 16
    WIN = 4000
    WIN2 = 2344
    KC = K // 32
    WC = 2000

    ws = 2.0 / jnp.asarray(resolution, jnp.float32)
    idx_cm = cube_idx.T.reshape(-1)
    voxf = voxel_coords.T.reshape(-1)
    sdf_p = jnp.pad(sdf, (0, MP - M_))
    defst = (deform * (jnp.asarray(resolution, jnp.float32) * 0.5)).T
    defst_p = jnp.pad(defst, ((0, 0), (0, MP - M_)))

    mesh = plsc.VectorSubcoreMesh(core_axis_name="c", subcore_axis_name="s")
    scp = pltpu.CompilerParams(needs_layout_passes=False,
                               use_tc_tiling_on_sc=False)

    scan = functools.partial(
        pl.kernel,
        mesh=mesh,
        compiler_params=scp,
        out_type=jax.ShapeDtypeStruct((2 * 16 * RANGE,), jnp.int32),
        scratch_types=[
            pltpu.VMEM((RANGE,), jnp.int32),
            pltpu.VMEM((WIN,), jnp.int32),
        ],
    )(functools.partial(_scan_kernel, N=N_, RANGE=RANGE, WIN=WIN))
    w_halves = scan(idx_cm)

    merge = functools.partial(
        pl.kernel,
        mesh=mesh,
        compiler_params=scp,
        out_type=jax.ShapeDtypeStruct((MP, 4), jnp.float32),
        scratch_types=[
            pltpu.VMEM((WIN2,), jnp.int32),
            pltpu.VMEM((WIN2,), jnp.int32),
            pltpu.VMEM((WIN2,), jnp.int32),
            pltpu.VMEM((3 * WIN2,), jnp.int32),
            pltpu.VMEM((3 * WIN2,), jnp.int32),
            pltpu.VMEM((WIN2,), jnp.float32),
            pltpu.VMEM((3 * WIN2,), jnp.float32),
            pltpu.VMEM((WIN2, 4), jnp.float32),
            pltpu.SemaphoreType.DMA,
        ],
    )(functools.partial(_merge_kernel, N=N_, RANGE=RB2 * 2, RB2=RB2,
                        WIN2=WIN2, NV=WIN2 // 16))
    table = merge(w_halves, voxf, sdf_p, defst_p)

    gather = functools.partial(
        pl.kernel,
        mesh=mesh,
        compiler_params=scp,
        out_type=jax.ShapeDtypeStruct((4, K), jnp.float32),
        scratch_types=[
            pltpu.VMEM((WC,), jnp.int32),
            pltpu.VMEM((WC, 4), jnp.float32),
            pltpu.VMEM((WC,), jnp.float32),
            pltpu.VMEM((WC,), jnp.float32),
            pltpu.VMEM((WC,), jnp.float32),
            pltpu.VMEM((WC,), jnp.float32),
            pltpu.SemaphoreType.DMA,
        ],
    )(functools.partial(_gather_kernel, KC=KC, WC=WC))
    g4 = gather(idx_cm, table)

    B = 2048
    out_t = pl.pallas_call(
        _dense_kernel,
        grid=(pl.cdiv(N_, B),),
        in_specs=[
            pl.BlockSpec((32, B), lambda i: (0, i)),
            pl.BlockSpec((12, B), lambda i: (0, i)),
            pl.BlockSpec((8, B), lambda i: (0, i)),
            pl.BlockSpec((1, B), lambda i: (0, i)),
            pl.BlockSpec((8, 128), lambda i: (0, 0)),
        ],
        out_specs=pl.BlockSpec((3, B), lambda i: (0, i)),
        out_shape=jax.ShapeDtypeStruct((3, N_), jnp.float32),
    )(g4.reshape(32, N_), beta.T, alpha.T,
      gamma.reshape(1, N_),
      jnp.broadcast_to(ws.reshape(1, 1), (8, 128)))
    return out_t.T


# speedup vs baseline: 21.2908x; 1.0518x over previous
import functools

import jax
import jax.numpy as jnp
from jax import lax
from jax.experimental import pallas as pl
from jax.experimental.pallas import tpu as pltpu
from jax.experimental.pallas import tpu_sc as plsc

_EA = (0, 1, 4, 0, 2, 3, 6, 2, 2, 3, 7, 6)
_EB = (1, 5, 5, 4, 3, 7, 7, 6, 0, 1, 5, 4)


def _scan_kernel(idx_hbm, w_out, w_v, idxwin_v, N, RANGE, WIN):
    h = lax.axis_index("c")
    r = lax.axis_index("s")
    lo = r * RANGE
    KH = N * 4
    nwin = KH // WIN

    def init_body(i, _):
        w_v[pl.ds(i * 16, 16)] = jnp.full((16,), -1, jnp.int32)
        return _

    lax.fori_loop(0, RANGE // 16, init_body, 0, unroll=4)

    iota = lax.iota(jnp.int32, 16)

    def win_body(w, _):
        base = h * KH + w * WIN
        pltpu.sync_copy(idx_hbm.at[pl.ds(base, WIN)], idxwin_v)
        c = base // N
        inplane = base - c * N
        kv8 = (inplane + iota) * 8 + c

        def v_body(i, _):
            iv = idxwin_v[pl.ds(i * 16, 16)]
            off = iv - lo
            m = (off >= 0) & (off < RANGE)
            cur = plsc.load_gather(w_v, [off], mask=m)
            kv = kv8 + i * 128
            m2 = m & (kv > cur)
            plsc.store_scatter(w_v, [off], kv, mask=m2)
            return _

        lax.fori_loop(0, WIN // 16, v_body, 0, unroll=8)
        return _

    lax.fori_loop(0, nwin, win_body, 0)
    pltpu.sync_copy(w_v, w_out.at[pl.ds((h * 16 + r) * RANGE, RANGE)])


def _merge_kernel(w_halves, voxf_hbm, sdf_hbm, defst_hbm, table_out,
                  w0_v, w1_v, kbuf_v, iv_v, voxcols_v, sdfwin_v, defwin_v,
                  out_v, sem, N, RANGE, RB2, WIN2, NV):
    t = lax.axis_index("s") * 2 + lax.axis_index("c")
    r = t // 2
    iota = lax.iota(jnp.int32, 16)
    zeros16 = jnp.zeros((16,), jnp.int32)
    ones16 = jnp.full((16,), 1, jnp.int32)
    twos16 = jnp.full((16,), 2, jnp.int32)
    threes16 = jnp.full((16,), 3, jnp.int32)

    def win_body(w, _):
        off_in_range = (t % 2) * RB2 + w * WIN2
        row_base = r * RANGE + off_in_range
        pltpu.sync_copy(w_halves.at[pl.ds(r * RANGE + off_in_range, WIN2)],
                        w0_v)
        pltpu.sync_copy(
            w_halves.at[pl.ds((16 + r) * RANGE + off_in_range, WIN2)], w1_v)
        pltpu.sync_copy(sdf_hbm.at[pl.ds(row_base, WIN2)], sdfwin_v)
        for comp in range(3):
            pltpu.sync_copy(defst_hbm.at[comp, pl.ds(row_base, WIN2)],
                            defwin_v.at[pl.ds(comp * WIN2, WIN2)])

        def merge_body(i, _):
            k = jnp.maximum(w0_v[pl.ds(i * 16, 16)], w1_v[pl.ds(i * 16, 16)])
            kbuf_v[pl.ds(i * 16, 16)] = k
            i_safe = jnp.where(k >= 0, lax.shift_right_logical(k, 3), 0)
            iv_v[pl.ds(i * 16, 16)] = i_safe
            iv_v[pl.ds(WIN2 + i * 16, 16)] = i_safe + N
            iv_v[pl.ds(2 * WIN2 + i * 16, 16)] = i_safe + 2 * N
            return _

        lax.fori_loop(0, NV, merge_body, 0, unroll=8)
        pltpu.async_copy(voxf_hbm.at[iv_v], voxcols_v, sem).wait()

        def build_body(i, _):
            rows = i * 16 + iota
            k = kbuf_v[pl.ds(i * 16, 16)]
            valid = k >= 0
            vx = voxcols_v[pl.ds(i * 16, 16)]
            vy = voxcols_v[pl.ds(WIN2 + i * 16, 16)]
            vz = voxcols_v[pl.ds(2 * WIN2 + i * 16, 16)]
            px = jnp.where(valid, (vx + (k & 1)).astype(jnp.float32), 0.0)
            py = jnp.where(
                valid,
                (vy + (lax.shift_right_logical(k, 1) & 1)).astype(jnp.float32),
                0.0)
            pz = jnp.where(
                valid,
                (vz + (lax.shift_right_logical(k, 2) & 1)).astype(jnp.float32),
                0.0)
            dx = defwin_v[pl.ds(i * 16, 16)]
            dy = defwin_v[pl.ds(WIN2 + i * 16, 16)]
            dz = defwin_v[pl.ds(2 * WIN2 + i * 16, 16)]
            s = sdfwin_v[pl.ds(i * 16, 16)]
            plsc.store_scatter(out_v, [rows, zeros16], px + dx)
            plsc.store_scatter(out_v, [rows, ones16], py + dy)
            plsc.store_scatter(out_v, [rows, twos16], pz + dz)
            plsc.store_scatter(out_v, [rows, threes16], s)
            return _

        lax.fori_loop(0, NV, build_body, 0, unroll=4)
        pltpu.sync_copy(out_v, table_out.at[pl.ds(row_base, WIN2)])
        return _

    lax.fori_loop(0, RB2 // WIN2, win_body, 0)


def _gather_kernel(idx_hbm, table_hbm, g_out, idxwin_v, rows_v,
                   t0_v, t1_v, t2_v, t3_v, sem, KC, WC):
    t = lax.axis_index("s") * 2 + lax.axis_index("c")
    iota = lax.iota(jnp.int32, 16)
    comps = (jnp.zeros((16,), jnp.int32), jnp.full((16,), 1, jnp.int32),
             jnp.full((16,), 2, jnp.int32), jnp.full((16,), 3, jnp.int32))
    tbufs = (t0_v, t1_v, t2_v, t3_v)

    def win_body(w, _):
        base = t * KC + w * WC
        pltpu.sync_copy(idx_hbm.at[pl.ds(base, WC)], idxwin_v)
        pltpu.async_copy(table_hbm.at[idxwin_v], rows_v, sem).wait()

        def tr_body(i, _):
            rows = i * 16 + iota
            for comp in range(4):
                tbufs[comp][pl.ds(i * 16, 16)] = plsc.load_gather(
                    rows_v, [rows, comps[comp]])
            return _

        lax.fori_loop(0, WC // 16, tr_body, 0, unroll=4)
        for comp in range(4):
            pltpu.sync_copy(tbufs[comp], g_out.at[comp, pl.ds(base, WC)])
        return _

    lax.fori_loop(0, KC // WC, win_body, 0)


def _dense_kernel(g4_ref, bt_ref, at_ref, gm_ref, ws_ref, out_ref):
    ws = ws_ref[0, 0]
    s = [g4_ref[24 + c, :] for c in range(8)]
    px = [(g4_ref[0 + c, :] + 0.5) * ws - 1.0 for c in range(8)]
    py = [(g4_ref[8 + c, :] + 0.5) * ws - 1.0 for c in range(8)]
    pz = [(g4_ref[16 + c, :] + 0.5) * ws - 1.0 for c in range(8)]

    occ_sum = sum((sc < 0).astype(jnp.float32) for sc in s)
    surf = ((occ_sum > 0) & (occ_sum < 8)).astype(jnp.float32)

    wsc = 0.99
    alpha_n = [jnp.tanh(at_ref[c, :]) * wsc + 1.0 for c in range(8)]
    gm = gm_ref[0, :]
    gamma_n = (1.0 / (1.0 + jnp.exp(-gm))) * wsc + (1.0 - wsc) / 2.0

    nx = jnp.zeros_like(gm)
    ny = jnp.zeros_like(gm)
    nz = jnp.zeros_like(gm)
    bw_sum = jnp.zeros_like(gm)
    for e in range(12):
        beta_n = jnp.tanh(bt_ref[e, :]) * wsc + 1.0
        sa = s[_EA[e]]
        sb = s[_EB[e]]
        denom = sa - sb
        denom = jnp.where(jnp.abs(denom) < 1e-8, 1e-8, denom)
        te = jnp.clip(sa / denom, 0.0, 1.0)
        crossing = ((sa * sb) < 0).astype(jnp.float32)
        bw = beta_n * crossing + 1e-6
        bw_sum = bw_sum + bw
        pax, pbx = px[_EA[e]], px[_EB[e]]
        pay, pby = py[_EA[e]], py[_EB[e]]
        paz, pbz = pz[_EA[e]], pz[_EB[e]]
        nx = nx + bw * (pax + te * (pbx - pax))
        ny = ny + bw * (pay + te * (pby - pay))
        nz = nz + bw * (paz + te * (pbz - paz))
    vex = nx / bw_sum
    vey = ny / bw_sum
    vez = nz / bw_sum

    a_sum = sum(alpha_n)
    vcx = sum(alpha_n[c] * px[c] for c in range(8)) / a_sum
    vcy = sum(alpha_n[c] * py[c] for c in range(8)) / a_sum
    vcz = sum(alpha_n[c] * pz[c] for c in range(8)) / a_sum

    out_ref[0, :] = (gamma_n * vex + (1.0 - gamma_n) * vcx) * surf
    out_ref[1, :] = (gamma_n * vey + (1.0 - gamma_n) * vcy) * surf
    out_ref[2, :] = (gamma_n * vez + (1.0 - gamma_n) * vcz) * surf


def kernel(voxel_coords, sdf, cube_idx, resolution, deform, beta, alpha, gamma):
    N_ = cube_idx.shape[0]
    M_ = sdf.shape[0]
    K = N_ * 8

    RB2 = ((M_ + 32 * 16 - 1) // (32 * 16)) * 16
    MP = RB2 * 32
    RANGE = MP // 16
    WIN = 20000
    WIN2 = 2344
    KC = K // 32
    WC = 2000

    ws = 2.0 / jnp.asarray(resolution, jnp.float32)
    idx_cm = cube_idx.T.reshape(-1)
    voxf = voxel_coords.T.reshape(-1)
    sdf_p = jnp.pad(sdf, (0, MP - M_))
    defst = (deform * (jnp.asarray(resolution, jnp.float32) * 0.5)).T
    defst_p = jnp.pad(defst, ((0, 0), (0, MP - M_)))

    mesh = plsc.VectorSubcoreMesh(core_axis_name="c", subcore_axis_name="s")
    scp = pltpu.CompilerParams(needs_layout_passes=False,
                               use_tc_tiling_on_sc=False)

    scan = functools.partial(
        pl.kernel,
        mesh=mesh,
        compiler_params=scp,
        out_type=jax.ShapeDtypeStruct((2 * 16 * RANGE,), jnp.int32),
        scratch_types=[
            pltpu.VMEM((RANGE,), jnp.int32),
            pltpu.VMEM((WIN,), jnp.int32),
        ],
    )(functools.partial(_scan_kernel, N=N_, RANGE=RANGE, WIN=WIN))
    w_halves = scan(idx_cm)

    merge = functools.partial(
        pl.kernel,
        mesh=mesh,
        compiler_params=scp,
        out_type=jax.ShapeDtypeStruct((MP, 4), jnp.float32),
        scratch_types=[
            pltpu.VMEM((WIN2,), jnp.int32),
            pltpu.VMEM((WIN2,), jnp.int32),
            pltpu.VMEM((WIN2,), jnp.int32),
            pltpu.VMEM((3 * WIN2,), jnp.int32),
            pltpu.VMEM((3 * WIN2,), jnp.int32),
            pltpu.VMEM((WIN2,), jnp.float32),
            pltpu.VMEM((3 * WIN2,), jnp.float32),
            pltpu.VMEM((WIN2, 4), jnp.float32),
            pltpu.SemaphoreType.DMA,
        ],
    )(functools.partial(_merge_kernel, N=N_, RANGE=RB2 * 2, RB2=RB2,
                        WIN2=WIN2, NV=WIN2 // 16))
    table = merge(w_halves, voxf, sdf_p, defst_p)

    gather = functools.partial(
        pl.kernel,
        mesh=mesh,
        compiler_params=scp,
        out_type=jax.ShapeDtypeStruct((4, K), jnp.float32),
        scratch_types=[
            pltpu.VMEM((WC,), jnp.int32),
            pltpu.VMEM((WC, 4), jnp.float32),
            pltpu.VMEM((WC,), jnp.float32),
            pltpu.VMEM((WC,), jnp.float32),
            pltpu.VMEM((WC,), jnp.float32),
            pltpu.VMEM((WC,), jnp.float32),
            pltpu.SemaphoreType.DMA,
        ],
    )(functools.partial(_gather_kernel, KC=KC, WC=WC))
    g4 = gather(idx_cm, table)

    B = 2048
    out_t = pl.pallas_call(
        _dense_kernel,
        grid=(pl.cdiv(N_, B),),
        in_specs=[
            pl.BlockSpec((32, B), lambda i: (0, i)),
            pl.BlockSpec((12, B), lambda i: (0, i)),
            pl.BlockSpec((8, B), lambda i: (0, i)),
            pl.BlockSpec((1, B), lambda i: (0, i)),
            pl.BlockSpec((8, 128), lambda i: (0, 0)),
        ],
        out_specs=pl.BlockSpec((3, B), lambda i: (0, i)),
        out_shape=jax.ShapeDtypeStruct((3, N_), jnp.float32),
    )(g4.reshape(32, N_), beta.T, alpha.T,
      gamma.reshape(1, N_),
      jnp.broadcast_to(ws.reshape(1, 1), (8, 128)))
    return out_t.T


# speedup vs baseline: 21.3369x; 1.0022x over previous
import functools

import jax
import jax.numpy as jnp
from jax import lax
from jax.experimental import pallas as pl
from jax.experimental.pallas import tpu as pltpu
from jax.experimental.pallas import tpu_sc as plsc

_EA = (0, 1, 4, 0, 2, 3, 6, 2, 2, 3, 7, 6)
_EB = (1, 5, 5, 4, 3, 7, 7, 6, 0, 1, 5, 4)


def _scan_kernel(idx_hbm, w_out, w_v, idxwin_v, N, RANGE, WIN):
    h = lax.axis_index("c")
    r = lax.axis_index("s")
    lo = r * RANGE
    KH = N * 4
    nwin = KH // WIN

    def init_body(i, _):
        w_v[pl.ds(i * 16, 16)] = jnp.full((16,), -1, jnp.int32)
        return _

    lax.fori_loop(0, RANGE // 16, init_body, 0, unroll=4)

    iota = lax.iota(jnp.int32, 16)

    def win_body(w, _):
        base = h * KH + w * WIN
        pltpu.sync_copy(idx_hbm.at[pl.ds(base, WIN)], idxwin_v)
        c = base // N
        inplane = base - c * N
        kv8 = (inplane + iota) * 8 + c

        def v_body(i, _):
            iv = idxwin_v[pl.ds(i * 16, 16)]
            off = iv - lo
            m = plsc.bitcast(off, jnp.uint32) < jnp.uint32(RANGE)
            cur = plsc.load_gather(w_v, [off], mask=m)
            kv = kv8 + i * 128
            m2 = m & (kv > cur)
            plsc.store_scatter(w_v, [off], kv, mask=m2)
            return _

        lax.fori_loop(0, WIN // 16, v_body, 0, unroll=16)
        return _

    lax.fori_loop(0, nwin, win_body, 0)
    pltpu.sync_copy(w_v, w_out.at[pl.ds((h * 16 + r) * RANGE, RANGE)])


def _merge_kernel(w_halves, voxf_hbm, sdf_hbm, defst_hbm, table_out,
                  w0_v, w1_v, kbuf_v, iv_v, voxcols_v, sdfwin_v, defwin_v,
                  out_v, sem, N, RANGE, RB2, WIN2, NV):
    t = lax.axis_index("s") * 2 + lax.axis_index("c")
    r = t // 2
    iota = lax.iota(jnp.int32, 16)
    zeros16 = jnp.zeros((16,), jnp.int32)
    ones16 = jnp.full((16,), 1, jnp.int32)
    twos16 = jnp.full((16,), 2, jnp.int32)
    threes16 = jnp.full((16,), 3, jnp.int32)

    def win_body(w, _):
        off_in_range = (t % 2) * RB2 + w * WIN2
        row_base = r * RANGE + off_in_range
        pltpu.sync_copy(w_halves.at[pl.ds(r * RANGE + off_in_range, WIN2)],
                        w0_v)
        pltpu.sync_copy(
            w_halves.at[pl.ds((16 + r) * RANGE + off_in_range, WIN2)], w1_v)
        pltpu.sync_copy(sdf_hbm.at[pl.ds(row_base, WIN2)], sdfwin_v)
        for comp in range(3):
            pltpu.sync_copy(defst_hbm.at[comp, pl.ds(row_base, WIN2)],
                            defwin_v.at[pl.ds(comp * WIN2, WIN2)])

        def merge_body(i, _):
            k = jnp.maximum(w0_v[pl.ds(i * 16, 16)], w1_v[pl.ds(i * 16, 16)])
            kbuf_v[pl.ds(i * 16, 16)] = k
            i_safe = jnp.where(k >= 0, lax.shift_right_logical(k, 3), 0)
            iv_v[pl.ds(i * 16, 16)] = i_safe
            iv_v[pl.ds(WIN2 + i * 16, 16)] = i_safe + N
            iv_v[pl.ds(2 * WIN2 + i * 16, 16)] = i_safe + 2 * N
            return _

        lax.fori_loop(0, NV, merge_body, 0, unroll=8)
        pltpu.async_copy(voxf_hbm.at[iv_v], voxcols_v, sem).wait()

        def build_body(i, _):
            rows = i * 16 + iota
            k = kbuf_v[pl.ds(i * 16, 16)]
            valid = k >= 0
            vx = voxcols_v[pl.ds(i * 16, 16)]
            vy = voxcols_v[pl.ds(WIN2 + i * 16, 16)]
            vz = voxcols_v[pl.ds(2 * WIN2 + i * 16, 16)]
            px = jnp.where(valid, (vx + (k & 1)).astype(jnp.float32), 0.0)
            py = jnp.where(
                valid,
                (vy + (lax.shift_right_logical(k, 1) & 1)).astype(jnp.float32),
                0.0)
            pz = jnp.where(
                valid,
                (vz + (lax.shift_right_logical(k, 2) & 1)).astype(jnp.float32),
                0.0)
            dx = defwin_v[pl.ds(i * 16, 16)]
            dy = defwin_v[pl.ds(WIN2 + i * 16, 16)]
            dz = defwin_v[pl.ds(2 * WIN2 + i * 16, 16)]
            s = sdfwin_v[pl.ds(i * 16, 16)]
            plsc.store_scatter(out_v, [rows, zeros16], px + dx)
            plsc.store_scatter(out_v, [rows, ones16], py + dy)
            plsc.store_scatter(out_v, [rows, twos16], pz + dz)
            plsc.store_scatter(out_v, [rows, threes16], s)
            return _

        lax.fori_loop(0, NV, build_body, 0, unroll=4)
        pltpu.sync_copy(out_v, table_out.at[pl.ds(row_base, WIN2)])
        return _

    lax.fori_loop(0, RB2 // WIN2, win_body, 0)


def _gather_kernel(idx_hbm, table_hbm, g_out, idxwin_v, rows_v,
                   t0_v, t1_v, t2_v, t3_v, sem, KC, WC):
    t = lax.axis_index("s") * 2 + lax.axis_index("c")
    iota = lax.iota(jnp.int32, 16)
    comps = (jnp.zeros((16,), jnp.int32), jnp.full((16,), 1, jnp.int32),
             jnp.full((16,), 2, jnp.int32), jnp.full((16,), 3, jnp.int32))
    tbufs = (t0_v, t1_v, t2_v, t3_v)

    def win_body(w, _):
        base = t * KC + w * WC
        pltpu.sync_copy(idx_hbm.at[pl.ds(base, WC)], idxwin_v)
        pltpu.async_copy(table_hbm.at[idxwin_v], rows_v, sem).wait()

        def tr_body(i, _):
            rows = i * 16 + iota
            for comp in range(4):
                tbufs[comp][pl.ds(i * 16, 16)] = plsc.load_gather(
                    rows_v, [rows, comps[comp]])
            return _

        lax.fori_loop(0, WC // 16, tr_body, 0, unroll=4)
        for comp in range(4):
            pltpu.sync_copy(tbufs[comp], g_out.at[comp, pl.ds(base, WC)])
        return _

    lax.fori_loop(0, KC // WC, win_body, 0)


def _dense_kernel(g4_ref, bt_ref, at_ref, gm_ref, ws_ref, out_ref):
    ws = ws_ref[0, 0]
    s = [g4_ref[24 + c, :] for c in range(8)]
    px = [(g4_ref[0 + c, :] + 0.5) * ws - 1.0 for c in range(8)]
    py = [(g4_ref[8 + c, :] + 0.5) * ws - 1.0 for c in range(8)]
    pz = [(g4_ref[16 + c, :] + 0.5) * ws - 1.0 for c in range(8)]

    occ_sum = sum((sc < 0).astype(jnp.float32) for sc in s)
    surf = ((occ_sum > 0) & (occ_sum < 8)).astype(jnp.float32)

    wsc = 0.99
    alpha_n = [jnp.tanh(at_ref[c, :]) * wsc + 1.0 for c in range(8)]
    gm = gm_ref[0, :]
    gamma_n = (1.0 / (1.0 + jnp.exp(-gm))) * wsc + (1.0 - wsc) / 2.0

    nx = jnp.zeros_like(gm)
    ny = jnp.zeros_like(gm)
    nz = jnp.zeros_like(gm)
    bw_sum = jnp.zeros_like(gm)
    for e in range(12):
        beta_n = jnp.tanh(bt_ref[e, :]) * wsc + 1.0
        sa = s[_EA[e]]
        sb = s[_EB[e]]
        denom = sa - sb
        denom = jnp.where(jnp.abs(denom) < 1e-8, 1e-8, denom)
        te = jnp.clip(sa / denom, 0.0, 1.0)
        crossing = ((sa * sb) < 0).astype(jnp.float32)
        bw = beta_n * crossing + 1e-6
        bw_sum = bw_sum + bw
        pax, pbx = px[_EA[e]], px[_EB[e]]
        pay, pby = py[_EA[e]], py[_EB[e]]
        paz, pbz = pz[_EA[e]], pz[_EB[e]]
        nx = nx + bw * (pax + te * (pbx - pax))
        ny = ny + bw * (pay + te * (pby - pay))
        nz = nz + bw * (paz + te * (pbz - paz))
    vex = nx / bw_sum
    vey = ny / bw_sum
    vez = nz / bw_sum

    a_sum = sum(alpha_n)
    vcx = sum(alpha_n[c] * px[c] for c in range(8)) / a_sum
    vcy = sum(alpha_n[c] * py[c] for c in range(8)) / a_sum
    vcz = sum(alpha_n[c] * pz[c] for c in range(8)) / a_sum

    out_ref[0, :] = (gamma_n * vex + (1.0 - gamma_n) * vcx) * surf
    out_ref[1, :] = (gamma_n * vey + (1.0 - gamma_n) * vcy) * surf
    out_ref[2, :] = (gamma_n * vez + (1.0 - gamma_n) * vcz) * surf


def kernel(voxel_coords, sdf, cube_idx, resolution, deform, beta, alpha, gamma):
    N_ = cube_idx.shape[0]
    M_ = sdf.shape[0]
    K = N_ * 8

    RB2 = ((M_ + 32 * 16 - 1) // (32 * 16)) * 16
    MP = RB2 * 32
    RANGE = MP // 16
    WIN = 20000
    WIN2 = 2344
    KC = K // 32
    WC = 2000

    ws = 2.0 / jnp.asarray(resolution, jnp.float32)
    idx_cm = cube_idx.T.reshape(-1)
    voxf = voxel_coords.T.reshape(-1)
    sdf_p = jnp.pad(sdf, (0, MP - M_))
    defst = (deform * (jnp.asarray(resolution, jnp.float32) * 0.5)).T
    defst_p = jnp.pad(defst, ((0, 0), (0, MP - M_)))

    mesh = plsc.VectorSubcoreMesh(core_axis_name="c", subcore_axis_name="s")
    scp = pltpu.CompilerParams(needs_layout_passes=False,
                               use_tc_tiling_on_sc=False)

    scan = functools.partial(
        pl.kernel,
        mesh=mesh,
        compiler_params=scp,
        out_type=jax.ShapeDtypeStruct((2 * 16 * RANGE,), jnp.int32),
        scratch_types=[
            pltpu.VMEM((RANGE,), jnp.int32),
            pltpu.VMEM((WIN,), jnp.int32),
        ],
    )(functools.partial(_scan_kernel, N=N_, RANGE=RANGE, WIN=WIN))
    w_halves = scan(idx_cm)

    merge = functools.partial(
        pl.kernel,
        mesh=mesh,
        compiler_params=scp,
        out_type=jax.ShapeDtypeStruct((MP, 4), jnp.float32),
        scratch_types=[
            pltpu.VMEM((WIN2,), jnp.int32),
            pltpu.VMEM((WIN2,), jnp.int32),
            pltpu.VMEM((WIN2,), jnp.int32),
            pltpu.VMEM((3 * WIN2,), jnp.int32),
            pltpu.VMEM((3 * WIN2,), jnp.int32),
            pltpu.VMEM((WIN2,), jnp.float32),
            pltpu.VMEM((3 * WIN2,), jnp.float32),
            pltpu.VMEM((WIN2, 4), jnp.float32),
            pltpu.SemaphoreType.DMA,
        ],
    )(functools.partial(_merge_kernel, N=N_, RANGE=RB2 * 2, RB2=RB2,
                        WIN2=WIN2, NV=WIN2 // 16))
    table = merge(w_halves, voxf, sdf_p, defst_p)

    gather = functools.partial(
        pl.kernel,
        mesh=mesh,
        compiler_params=scp,
        out_type=jax.ShapeDtypeStruct((4, K), jnp.float32),
        scratch_types=[
            pltpu.VMEM((WC,), jnp.int32),
            pltpu.VMEM((WC, 4), jnp.float32),
            pltpu.VMEM((WC,), jnp.float32),
            pltpu.VMEM((WC,), jnp.float32),
            pltpu.VMEM((WC,), jnp.float32),
            pltpu.VMEM((WC,), jnp.float32),
            pltpu.SemaphoreType.DMA,
        ],
    )(functools.partial(_gather_kernel, KC=KC, WC=WC))
    g4 = gather(idx_cm, table)

    B = 2048
    out_t = pl.pallas_call(
        _dense_kernel,
        grid=(pl.cdiv(N_, B),),
        in_specs=[
            pl.BlockSpec((32, B), lambda i: (0, i)),
            pl.BlockSpec((12, B), lambda i: (0, i)),
            pl.BlockSpec((8, B), lambda i: (0, i)),
            pl.BlockSpec((1, B), lambda i: (0, i)),
            pl.BlockSpec((8, 128), lambda i: (0, 0)),
        ],
        out_specs=pl.BlockSpec((3, B), lambda i: (0, i)),
        out_shape=jax.ShapeDtypeStruct((3, N_), jnp.float32),
    )(g4.reshape(32, N_), beta.T, alpha.T,
      gamma.reshape(1, N_),
      jnp.broadcast_to(ws.reshape(1, 1), (8, 128)))
    return out_t.T
